# baseline (speedup 1.0000x reference)
import functools

import jax
import jax.numpy as jnp
from jax import lax
from jax.experimental import pallas as pl
from jax.experimental.pallas import tpu as pltpu
from jax.experimental.pallas import tpu_sc as plsc

N_NODES = 10000
N_PAD = 10240
D = 128
N_EDGES = 320000
NC, NS, L = 2, 16, 16
NW = NC * NS
E_PAD = 327680
CHUNK = 128
N_CHUNKS = E_PAD // CHUNK
TILE_CHUNKS = N_CHUNKS // NW
Q_CHUNKS = TILE_CHUNKS // 4
ROWS_PER_TILE = N_PAD // NS
ACC_ROWS_PER_TILE = N_NODES // NS
NBUF = 2

_mesh = plsc.VectorSubcoreMesh(
    core_axis_name="c", subcore_axis_name="s", num_cores=NC, num_subcores=NS)


@functools.partial(
    pl.kernel,
    out_type=jax.ShapeDtypeStruct((NC, N_PAD), jnp.float32),
    mesh=_mesh,
    scratch_types=[
        pltpu.VMEM((TILE_CHUNKS, CHUNK), jnp.int32),
        pltpu.VMEM((TILE_CHUNKS, CHUNK), jnp.float32),
        pltpu.VMEM((ROWS_PER_TILE,), jnp.float32),
        pltpu.VMEM_SHARED((N_PAD,), jnp.float32),
        pltpu.SemaphoreType.DMA,
    ],
)
def _deg_kernel(col_hbm, ew_hbm, out_hbm, coli, ewi, zbuf, acc, sem):
    cid = lax.axis_index("c")
    sid = lax.axis_index("s")
    wid = cid * NS + sid
    cb = wid * TILE_CHUNKS

    zv = jnp.zeros((L,), jnp.float32)
    def _zero(i, _):
        zbuf[pl.ds(i * L, L)] = zv
        return 0
    lax.fori_loop(0, ROWS_PER_TILE // L, _zero, 0)
    pltpu.sync_copy(zbuf, acc.at[pl.ds(sid * ROWS_PER_TILE, ROWS_PER_TILE)])
    pltpu.sync_copy(col_hbm.at[pl.ds(cb, TILE_CHUNKS)], coli)
    pltpu.sync_copy(ew_hbm.at[pl.ds(cb, TILE_CHUNKS)], ewi)
    plsc.subcore_barrier()

    def _fire(j, _):
        pltpu.async_copy(ewi.at[j], acc.at[coli.at[j]], sem, add=True)
        return 0
    lax.fori_loop(0, TILE_CHUNKS, _fire, 0)
    pltpu.make_async_copy(ew_hbm.at[pl.ds(0, TILE_CHUNKS)], ewi, sem).wait()
    plsc.subcore_barrier()

    pltpu.sync_copy(
        acc.at[pl.ds(sid * ROWS_PER_TILE, ROWS_PER_TILE)],
        out_hbm.at[cid, pl.ds(sid * ROWS_PER_TILE, ROWS_PER_TILE)])


@functools.partial(
    pl.kernel,
    out_type=jax.ShapeDtypeStruct((NC, N_NODES, D), jnp.float32),
    mesh=_mesh,
    scratch_types=[
        pltpu.VMEM((Q_CHUNKS, CHUNK), jnp.int32),
        pltpu.VMEM((Q_CHUNKS, CHUNK), jnp.int32),
        pltpu.VMEM((Q_CHUNKS, CHUNK), jnp.float32),
        [pltpu.VMEM((CHUNK, D), jnp.float32)] * NBUF,
        pltpu.VMEM_SHARED((N_NODES, D), jnp.float32),
        [pltpu.SemaphoreType.DMA] * NBUF,
        [pltpu.SemaphoreType.DMA] * NBUF,
    ],
    compiler_params=pltpu.CompilerParams(use_tc_tiling_on_sc=False),
)
def _agg_kernel(row_hbm, col_hbm, ew_hbm, hs_hbm, out_hbm,
                rowi, coli, ewi, gbufs, acc, gsems, ssems):
    cid = lax.axis_index("c")
    sid = lax.axis_index("s")
    wid = cid * NS + sid

    zv = jnp.zeros((L,), jnp.float32)
    def _zero(i, _):
        for k in range(D // L):
            gbufs[0][i, pl.ds(k * L, L)] = zv
        return 0
    lax.fori_loop(0, CHUNK, _zero, 0)
    for j in range(5):
        pltpu.sync_copy(
            gbufs[0].at[pl.ds(0, 125)],
            acc.at[pl.ds(sid * ACC_ROWS_PER_TILE + j * 125, 125)])
    plsc.subcore_barrier()

    def _drain(buf, sem):
        pltpu.make_async_copy(hs_hbm.at[pl.ds(0, CHUNK)], buf, sem).wait()

    def _scale(gb, j):
        def _group(g, _):
            ew16 = ewi[j, pl.ds(g * L, L)]
            for t in range(L):
                e = g * L + t
                sv = jnp.full((L,), ew16[t], jnp.float32)
                for k in range(D // L):
                    gb[e, pl.ds(k * L, L)] = gb[e, pl.ds(k * L, L)] * sv
            return 0
        lax.fori_loop(0, CHUNK // L, _group, 0)

    for h in range(4):
        cb = wid * TILE_CHUNKS + h * Q_CHUNKS
        pltpu.sync_copy(row_hbm.at[pl.ds(cb, Q_CHUNKS)], rowi)
        pltpu.sync_copy(col_hbm.at[pl.ds(cb, Q_CHUNKS)], coli)
        pltpu.sync_copy(ew_hbm.at[pl.ds(cb, Q_CHUNKS)], ewi)

        for i in range(NBUF):
            pltpu.async_copy(hs_hbm.at[rowi.at[i]], gbufs[i], gsems[i])

        def _step(t, _):
            for i in range(NBUF):
                jj = t * NBUF + i
                _drain(gbufs[i], gsems[i])
                ip = (i - 1) % NBUF
                @pl.when((jj >= 1) & (jj + NBUF - 1 < Q_CHUNKS))
                def _refill():
                    _drain(gbufs[ip], ssems[ip])
                    pltpu.async_copy(hs_hbm.at[rowi.at[jj + NBUF - 1]],
                                     gbufs[ip], gsems[ip])
                _scale(gbufs[i], jj)
                pltpu.async_copy(gbufs[i], acc.at[coli.at[jj]], ssems[i],
                                 add=True)
            return 0
        lax.fori_loop(0, Q_CHUNKS // NBUF, _step, 0)
        for i in range(NBUF):
            _drain(gbufs[i], ssems[i])
    plsc.subcore_barrier()

    for j in range(5):
        r = sid * ACC_ROWS_PER_TILE + j * 125
        pltpu.sync_copy(acc.at[pl.ds(r, 125)],
                        out_hbm.at[cid, pl.ds(r, 125)])


_BLK = 1000
_GRID = N_NODES // _BLK


def _dis_body(degp_ref, out_ref):
    deg = degp_ref[0] + degp_ref[1] + 1.0
    out_ref[...] = jnp.where(deg > 0, lax.rsqrt(deg), 0.0)


def _dis_tc(degp):
    shp = degp.shape[1:]
    return pl.pallas_call(
        _dis_body,
        out_shape=jax.ShapeDtypeStruct(shp, jnp.float32),
    )(degp)


def _mm_scale_body(x_ref, w_ref, dis_ref, out_ref):
    h = jnp.dot(x_ref[...], w_ref[...], preferred_element_type=jnp.float32)
    out_ref[...] = dis_ref[...] * h


def _mm_scale_tc(x, w, dis_b):
    return pl.pallas_call(
        _mm_scale_body,
        grid=(_GRID,),
        in_specs=[
            pl.BlockSpec((_BLK, D), lambda i: (i, 0)),
            pl.BlockSpec((D, D), lambda i: (0, 0)),
            pl.BlockSpec((_BLK, D), lambda i: (i, 0)),
        ],
        out_specs=pl.BlockSpec((_BLK, D), lambda i: (i, 0)),
        out_shape=jax.ShapeDtypeStruct((N_NODES, D), jnp.float32),
    )(x, w, dis_b)


def _combine_mm_body(raw_ref, hs_ref, dis_ref, b_ref, a_ref, w_ref, out_ref):
    raw = raw_ref[...]
    d = dis_ref[...]
    z = d * (raw[0] + raw[1] + hs_ref[...]) + b_ref[...]
    z = jnp.where(z > 0, z, a_ref[...] * z)
    h = jnp.dot(z, w_ref[...], preferred_element_type=jnp.float32)
    out_ref[...] = d * h


def _combine_mm_tc(raw, hs, dis_b, b, a, w):
    return pl.pallas_call(
        _combine_mm_body,
        grid=(_GRID,),
        in_specs=[
            pl.BlockSpec((NC, _BLK, D), lambda i: (0, i, 0)),
            pl.BlockSpec((_BLK, D), lambda i: (i, 0)),
            pl.BlockSpec((_BLK, D), lambda i: (i, 0)),
            pl.BlockSpec((1, D), lambda i: (0, 0)),
            pl.BlockSpec((1, D), lambda i: (0, 0)),
            pl.BlockSpec((D, D), lambda i: (0, 0)),
        ],
        out_specs=pl.BlockSpec((_BLK, D), lambda i: (i, 0)),
        out_shape=jax.ShapeDtypeStruct((N_NODES, D), jnp.float32),
    )(raw, hs, dis_b, b, a, w)


def _combine_body(raw_ref, hs_ref, dis_ref, b_ref, a_ref, out_ref):
    raw = raw_ref[...]
    z = dis_ref[...] * (raw[0] + raw[1] + hs_ref[...]) + b_ref[...]
    out_ref[...] = jnp.where(z > 0, z, a_ref[...] * z)


def _combine_tc(raw, hs, dis_b, b, a):
    return pl.pallas_call(
        _combine_body,
        grid=(_GRID,),
        in_specs=[
            pl.BlockSpec((NC, _BLK, D), lambda i: (0, i, 0)),
            pl.BlockSpec((_BLK, D), lambda i: (i, 0)),
            pl.BlockSpec((_BLK, D), lambda i: (i, 0)),
            pl.BlockSpec((1, D), lambda i: (0, 0)),
            pl.BlockSpec((1, D), lambda i: (0, 0)),
        ],
        out_specs=pl.BlockSpec((_BLK, D), lambda i: (i, 0)),
        out_shape=jax.ShapeDtypeStruct((N_NODES, D), jnp.float32),
    )(raw, hs, dis_b, b, a)


@jax.jit
def kernel(x, edge_index, edge_weight, W1, b1, a1, W2, b2, a2):
    row = edge_index[0].astype(jnp.int32)
    col = edge_index[1].astype(jnp.int32)
    ew = edge_weight.astype(jnp.float32)

    pad_e = E_PAD - N_EDGES
    row = jnp.concatenate([row, jnp.zeros((pad_e,), jnp.int32)])
    col = jnp.concatenate([col, jnp.zeros((pad_e,), jnp.int32)])
    ew = jnp.concatenate([ew, jnp.zeros((pad_e,), jnp.float32)])
    row2 = row.reshape(N_CHUNKS, CHUNK)
    col2 = col.reshape(N_CHUNKS, CHUNK)
    ew2 = ew.reshape(N_CHUNKS, CHUNK)

    xf = x.astype(jnp.float32)

    degp = _deg_kernel(col2, ew2)
    dis = _dis_tc(degp.reshape(NC, N_PAD // D, D))
    dis_b = jnp.broadcast_to(
        dis.reshape(N_PAD)[:N_NODES, None], (N_NODES, D))

    b1r = b1.reshape(1, D)
    a1r = a1.reshape(1, D)
    b2r = b2.reshape(1, D)
    a2r = a2.reshape(1, D)

    hs1 = _mm_scale_tc(xf, W1, dis_b)
    raw1 = _agg_kernel(row2, col2, ew2, hs1)
    hs2 = _combine_mm_tc(raw1, hs1, dis_b, b1r, a1r, W2)
    raw2 = _agg_kernel(row2, col2, ew2, hs2)
    out = _combine_tc(raw2, hs2, dis_b, b2r, a2r)
    return out


# speedup vs baseline: 1.7185x; 1.7185x over previous
import functools

import jax
import jax.numpy as jnp
from jax import lax
from jax.experimental import pallas as pl
from jax.experimental.pallas import tpu as pltpu
from jax.experimental.pallas import tpu_sc as plsc

N_NODES = 10000
N_PAD = 10240
D = 128
DH = D // 2
N_EDGES = 320000
NC, NS, L = 2, 16, 16
NW = NC * NS
E_PAD = 327680
CHUNK = 128
N_CHUNKS = E_PAD // CHUNK
TILE_CHUNKS = N_CHUNKS // NS
Q_CHUNKS = TILE_CHUNKS // 4
DEG_CHUNKS = N_CHUNKS // NW
ROWS_PER_TILE = N_PAD // NS
NBUF = 4

_mesh = plsc.VectorSubcoreMesh(
    core_axis_name="c", subcore_axis_name="s", num_cores=NC, num_subcores=NS)


@functools.partial(
    pl.kernel,
    out_type=jax.ShapeDtypeStruct((NC, N_PAD), jnp.float32),
    mesh=_mesh,
    scratch_types=[
        pltpu.VMEM((DEG_CHUNKS, CHUNK), jnp.int32),
        pltpu.VMEM((DEG_CHUNKS, CHUNK), jnp.float32),
        pltpu.VMEM((ROWS_PER_TILE,), jnp.float32),
        pltpu.VMEM_SHARED((N_PAD,), jnp.float32),
        pltpu.SemaphoreType.DMA,
    ],
)
def _deg_kernel(col_hbm, ew_hbm, out_hbm, coli, ewi, zbuf, acc, sem):
    cid = lax.axis_index("c")
    sid = lax.axis_index("s")
    wid = cid * NS + sid
    cb = wid * DEG_CHUNKS

    zv = jnp.zeros((L,), jnp.float32)
    def _zero(i, _):
        zbuf[pl.ds(i * L, L)] = zv
        return 0
    lax.fori_loop(0, ROWS_PER_TILE // L, _zero, 0)
    pltpu.sync_copy(zbuf, acc.at[pl.ds(sid * ROWS_PER_TILE, ROWS_PER_TILE)])
    pltpu.sync_copy(col_hbm.at[pl.ds(cb, DEG_CHUNKS)], coli)
    pltpu.sync_copy(ew_hbm.at[pl.ds(cb, DEG_CHUNKS)], ewi)
    plsc.subcore_barrier()

    def _fire(j, _):
        pltpu.async_copy(ewi.at[j], acc.at[coli.at[j]], sem, add=True)
        return 0
    lax.fori_loop(0, DEG_CHUNKS, _fire, 0)
    pltpu.make_async_copy(ew_hbm.at[pl.ds(0, DEG_CHUNKS)], ewi, sem).wait()
    plsc.subcore_barrier()

    pltpu.sync_copy(
        acc.at[pl.ds(sid * ROWS_PER_TILE, ROWS_PER_TILE)],
        out_hbm.at[cid, pl.ds(sid * ROWS_PER_TILE, ROWS_PER_TILE)])


@functools.partial(
    pl.kernel,
    out_type=jax.ShapeDtypeStruct((NC, N_PAD, DH), jnp.float32),
    mesh=_mesh,
    scratch_types=[
        pltpu.VMEM((Q_CHUNKS, CHUNK), jnp.int32),
        pltpu.VMEM((Q_CHUNKS, CHUNK), jnp.int32),
        pltpu.VMEM((Q_CHUNKS, CHUNK), jnp.float32),
        [pltpu.VMEM((CHUNK, DH), jnp.float32)] * NBUF,
        pltpu.VMEM_SHARED((N_PAD, DH), jnp.float32),
        pltpu.VMEM_SHARED((N_PAD, DH), jnp.float32),
        [pltpu.SemaphoreType.DMA] * NBUF,
        [pltpu.SemaphoreType.DMA] * NBUF,
    ],
    compiler_params=pltpu.CompilerParams(use_tc_tiling_on_sc=False),
)
def _agg_kernel(row_hbm, col_hbm, ew_hbm, hsf_hbm, out_hbm,
                rowi, coli, ewi, gbufs, hs_sp, acc, gsems, ssems):
    cid = lax.axis_index("c")
    sid = lax.axis_index("s")

    rb = sid * ROWS_PER_TILE
    pltpu.sync_copy(hsf_hbm.at[pl.ds(cid * N_PAD + rb, ROWS_PER_TILE)],
                    hs_sp.at[pl.ds(rb, ROWS_PER_TILE)])
    zv = jnp.zeros((L,), jnp.float32)
    def _zero(i, _):
        for k in range(DH // L):
            gbufs[0][i, pl.ds(k * L, L)] = zv
        return 0
    lax.fori_loop(0, CHUNK, _zero, 0)
    for j in range(ROWS_PER_TILE // CHUNK):
        pltpu.sync_copy(
            gbufs[0], acc.at[pl.ds(rb + j * CHUNK, CHUNK)])
    plsc.subcore_barrier()

    def _drain(buf, sem):
        pltpu.make_async_copy(hsf_hbm.at[pl.ds(0, CHUNK)], buf, sem).wait()

    def _scale(gb, j):
        def _group(g, _):
            ew16 = ewi[j, pl.ds(g * L, L)]
            for t in range(L):
                e = g * L + t
                sv = jnp.full((L,), ew16[t], jnp.float32)
                for k in range(DH // L):
                    gb[e, pl.ds(k * L, L)] = gb[e, pl.ds(k * L, L)] * sv
            return 0
        lax.fori_loop(0, CHUNK // L, _group, 0)

    def _from_hbm(i, h):
        return i < NBUF // 2 or (i == 2 and h == 0)

    def _gather(i, h, idxref, buf, sem):
        if _from_hbm(i, h):
            pltpu.async_copy(hsf_hbm.at[idxref], buf, sem)
        else:
            pltpu.async_copy(hs_sp.at[idxref], buf, sem)

    roff = jnp.full((L,), cid * N_PAD, jnp.int32)
    for h in range(4):
        cb = sid * TILE_CHUNKS + h * Q_CHUNKS
        pltpu.sync_copy(row_hbm.at[pl.ds(cb, Q_CHUNKS)], rowi)
        pltpu.sync_copy(col_hbm.at[pl.ds(cb, Q_CHUNKS)], coli)
        pltpu.sync_copy(ew_hbm.at[pl.ds(cb, Q_CHUNKS)], ewi)

        hbm_slots = [i for i in range(NBUF) if _from_hbm(i, h)]
        def _rebase(q, _):
            for i in hbm_slots:
                j = q * NBUF + i
                for g in range(CHUNK // L):
                    rowi[j, pl.ds(g * L, L)] = (
                        rowi[j, pl.ds(g * L, L)] + roff)
            return 0
        lax.fori_loop(0, Q_CHUNKS // NBUF, _rebase, 0)

        for i in range(NBUF):
            _gather(i, h, rowi.at[i], gbufs[i], gsems[i])

        def _step(t, _):
            for i in range(NBUF):
                jj = t * NBUF + i
                _drain(gbufs[i], gsems[i])
                _scale(gbufs[i], jj)
                pltpu.async_copy(gbufs[i], acc.at[coli.at[jj]], ssems[i],
                                 add=True)
                ip = (i - 1) % NBUF
                @pl.when((jj >= 1) & (jj + NBUF - 1 < Q_CHUNKS))
                def _refill():
                    _drain(gbufs[ip], ssems[ip])
                    _gather(ip, h, rowi.at[jj + NBUF - 1],
                            gbufs[ip], gsems[ip])
            return 0
        lax.fori_loop(0, Q_CHUNKS // NBUF, _step, 0)
        for i in range(NBUF):
            _drain(gbufs[i], ssems[i])
    plsc.subcore_barrier()

    for j in range(ROWS_PER_TILE // CHUNK):
        r = sid * ROWS_PER_TILE + j * CHUNK
        pltpu.sync_copy(acc.at[pl.ds(r, CHUNK)],
                        out_hbm.at[cid, pl.ds(r, CHUNK)])


_BLK = 1024
_GRID = N_PAD // _BLK


def _dis_body(degp_ref, out_ref):
    deg = degp_ref[0] + degp_ref[1] + 1.0
    out_ref[...] = jnp.where(deg > 0, lax.rsqrt(deg), 0.0)


def _dis_tc(degp):
    shp = degp.shape[1:]
    return pl.pallas_call(
        _dis_body,
        out_shape=jax.ShapeDtypeStruct(shp, jnp.float32),
    )(degp)


def _mm_scale_body(x_ref, w_ref, dis_ref, out_ref):
    h = jnp.dot(x_ref[...], w_ref[...], preferred_element_type=jnp.float32)
    v = dis_ref[...] * h
    out_ref[0] = v[:, :DH]
    out_ref[1] = v[:, DH:]


def _mm_scale_tc(x, w, dis_b):
    return pl.pallas_call(
        _mm_scale_body,
        grid=(_GRID,),
        in_specs=[
            pl.BlockSpec((_BLK, D), lambda i: (i, 0)),
            pl.BlockSpec((D, D), lambda i: (0, 0)),
            pl.BlockSpec((_BLK, D), lambda i: (i, 0)),
        ],
        out_specs=pl.BlockSpec((NC, _BLK, DH), lambda i: (0, i, 0)),
        out_shape=jax.ShapeDtypeStruct((NC, N_PAD, DH), jnp.float32),
    )(x, w, dis_b)


def _combine_mm_body(raw_ref, hs_ref, dis_ref, b_ref, a_ref, w_ref, out_ref):
    raw = raw_ref[...]
    hs = hs_ref[...]
    d = dis_ref[...]
    s = jnp.concatenate([raw[0] + hs[0], raw[1] + hs[1]], axis=1)
    z = d * s + b_ref[...]
    z = jnp.where(z > 0, z, a_ref[...] * z)
    h = jnp.dot(z, w_ref[...], preferred_element_type=jnp.float32)
    v = d * h
    out_ref[0] = v[:, :DH]
    out_ref[1] = v[:, DH:]


def _combine_mm_tc(raw, hs, dis_b, b, a, w):
    return pl.pallas_call(
        _combine_mm_body,
        grid=(_GRID,),
        in_specs=[
            pl.BlockSpec((NC, _BLK, DH), lambda i: (0, i, 0)),
            pl.BlockSpec((NC, _BLK, DH), lambda i: (0, i, 0)),
            pl.BlockSpec((_BLK, D), lambda i: (i, 0)),
            pl.BlockSpec((1, D), lambda i: (0, 0)),
            pl.BlockSpec((1, D), lambda i: (0, 0)),
            pl.BlockSpec((D, D), lambda i: (0, 0)),
        ],
        out_specs=pl.BlockSpec((NC, _BLK, DH), lambda i: (0, i, 0)),
        out_shape=jax.ShapeDtypeStruct((NC, N_PAD, DH), jnp.float32),
    )(raw, hs, dis_b, b, a, w)


def _combine_body(raw_ref, hs_ref, dis_ref, b_ref, a_ref, out_ref):
    raw = raw_ref[...]
    hs = hs_ref[...]
    s = jnp.concatenate([raw[0] + hs[0], raw[1] + hs[1]], axis=1)
    z = dis_ref[...] * s + b_ref[...]
    out_ref[...] = jnp.where(z > 0, z, a_ref[...] * z)


def _combine_tc(raw, hs, dis_b, b, a):
    return pl.pallas_call(
        _combine_body,
        grid=(_GRID,),
        in_specs=[
            pl.BlockSpec((NC, _BLK, DH), lambda i: (0, i, 0)),
            pl.BlockSpec((NC, _BLK, DH), lambda i: (0, i, 0)),
            pl.BlockSpec((_BLK, D), lambda i: (i, 0)),
            pl.BlockSpec((1, D), lambda i: (0, 0)),
            pl.BlockSpec((1, D), lambda i: (0, 0)),
        ],
        out_specs=pl.BlockSpec((_BLK, D), lambda i: (i, 0)),
        out_shape=jax.ShapeDtypeStruct((N_PAD, D), jnp.float32),
    )(raw, hs, dis_b, b, a)


@jax.jit
def kernel(x, edge_index, edge_weight, W1, b1, a1, W2, b2, a2):
    row = edge_index[0].astype(jnp.int32)
    col = edge_index[1].astype(jnp.int32)
    ew = edge_weight.astype(jnp.float32)

    pad_e = E_PAD - N_EDGES
    row = jnp.concatenate([row, jnp.zeros((pad_e,), jnp.int32)])
    col = jnp.concatenate([col, jnp.zeros((pad_e,), jnp.int32)])
    ew = jnp.concatenate([ew, jnp.zeros((pad_e,), jnp.float32)])
    row2 = row.reshape(N_CHUNKS, CHUNK)
    col2 = col.reshape(N_CHUNKS, CHUNK)
    ew2 = ew.reshape(N_CHUNKS, CHUNK)

    x_pad = jnp.concatenate(
        [x, jnp.zeros((N_PAD - N_NODES, D), x.dtype)]).astype(jnp.float32)

    degp = _deg_kernel(col2, ew2)
    dis = _dis_tc(degp.reshape(NC, N_PAD // D, D))
    dis_b = jnp.broadcast_to(dis.reshape(N_PAD)[:, None], (N_PAD, D))

    b1r = b1.reshape(1, D)
    a1r = a1.reshape(1, D)
    b2r = b2.reshape(1, D)
    a2r = a2.reshape(1, D)

    hs1 = _mm_scale_tc(x_pad, W1, dis_b)
    raw1 = _agg_kernel(row2, col2, ew2, hs1.reshape(NC * N_PAD, DH))
    hs2 = _combine_mm_tc(raw1, hs1, dis_b, b1r, a1r, W2)
    raw2 = _agg_kernel(row2, col2, ew2, hs2.reshape(NC * N_PAD, DH))
    out = _combine_tc(raw2, hs2, dis_b, b2r, a2r)
    return out[:N_NODES]
